# baseline (speedup 1.0000x reference)
import jax
import jax.numpy as jnp
from jax.experimental import pallas as pl
from jax.experimental.pallas import tpu as pltpu

N_EXPERTS = 7
K_PROJ = 6
QUANT_TEMP = 0.3
BLK = 512


def _route_block(zt, anchors, inv_2temp):
    mean = jnp.mean(zt, axis=0, keepdims=True)
    var = jnp.mean((zt - mean) * (zt - mean), axis=0, keepdims=True)
    scale = jnp.sqrt(var) + 1e-6
    qt = jnp.tanh(zt / (QUANT_TEMP * scale))

    dott = jax.lax.dot_general(
        anchors, qt, (((1,), (0,)), ((), ())),
        preferred_element_type=jnp.float32)

    logits = (dott - 6.0) * inv_2temp
    eidx = jax.lax.broadcasted_iota(jnp.int32, logits.shape, 0)

    m1 = jnp.max(logits, axis=0, keepdims=True)
    i1 = jnp.min(jnp.where(logits == m1, eidx, N_EXPERTS),
                 axis=0, keepdims=True)
    is1 = eidx == i1
    rest = jnp.where(is1, -jnp.inf, logits)
    m2 = jnp.max(rest, axis=0, keepdims=True)
    i2 = jnp.min(jnp.where(rest == m2, eidx, N_EXPERTS),
                 axis=0, keepdims=True)

    e2 = jnp.exp(m2 - m1)
    denom = 1.0 + e2
    w1 = 1.0 / denom
    w2 = e2 / denom
    ewt = jnp.where(is1, w1, 0.0) + jnp.where(eidx == i2, w2, 0.0)
    return ewt, qt


def _router_kernel(xa_ref, xb_ref, w_ref, a_ref, rt_ref,
                   ewa_ref, ewb_ref, qa_ref, qb_ref):
    w = w_ref[...]
    anchors = a_ref[...]
    inv_2temp = 1.0 / (2.0 * jnp.maximum(rt_ref[0], 0.1))

    zta = jax.lax.dot_general(
        w, xa_ref[...], (((1,), (1,)), ((), ())),
        preferred_element_type=jnp.float32)
    ewa_ref[...], qa_ref[...] = _route_block(zta, anchors, inv_2temp)

    ztb = jax.lax.dot_general(
        w, xb_ref[...], (((1,), (1,)), ((), ())),
        preferred_element_type=jnp.float32)
    ewb_ref[...], qb_ref[...] = _route_block(ztb, anchors, inv_2temp)


@jax.jit
def kernel(x, W_proj, routing_temp, expert_anchors):
    B, T, D = x.shape
    n_tok = B * T
    half = n_tok // 2
    x2 = x.reshape(n_tok, D)
    rt = routing_temp.astype(jnp.float32).reshape(1)

    n_blk = half // BLK
    grid = (n_blk,)
    ewa, ewb, qa, qb = pl.pallas_call(
        _router_kernel,
        grid=grid,
        in_specs=[
            pl.BlockSpec((BLK, D), lambda i: (i, 0)),
            pl.BlockSpec((BLK, D), lambda i, _n=n_blk: (i + _n, 0)),
            pl.BlockSpec((K_PROJ, D), lambda i: (0, 0)),
            pl.BlockSpec((N_EXPERTS, K_PROJ), lambda i: (0, 0)),
            pl.BlockSpec(memory_space=pltpu.SMEM),
        ],
        out_specs=[
            pl.BlockSpec((N_EXPERTS, BLK), lambda i: (0, i)),
            pl.BlockSpec((N_EXPERTS, BLK), lambda i: (0, i)),
            pl.BlockSpec((K_PROJ, BLK), lambda i: (0, i)),
            pl.BlockSpec((K_PROJ, BLK), lambda i: (0, i)),
        ],
        out_shape=[
            jax.ShapeDtypeStruct((N_EXPERTS, half), x.dtype),
            jax.ShapeDtypeStruct((N_EXPERTS, half), x.dtype),
            jax.ShapeDtypeStruct((K_PROJ, half), jnp.float32),
            jax.ShapeDtypeStruct((K_PROJ, half), jnp.float32),
        ],
        compiler_params=pltpu.CompilerParams(
            dimension_semantics=("arbitrary",)),
    )(x2, x2, W_proj, expert_anchors, rt)

    ewt = jnp.concatenate([ewa, ewb], axis=1)
    qt = jnp.concatenate([qa, qb], axis=1)
    return ewt.T.reshape(B, T, N_EXPERTS), qt.T.reshape(B, T, K_PROJ)


# speedup vs baseline: 1.0309x; 1.0309x over previous
import jax
import jax.numpy as jnp
from jax.experimental import pallas as pl
from jax.experimental.pallas import tpu as pltpu

N_EXPERTS = 7
K_PROJ = 6
QUANT_TEMP = 0.3
BLK = 512


def _router_kernel(x_ref, w_ref, a_ref, rt_ref, ew_ref, q_ref):
    x = x_ref[...]
    w = w_ref[...]
    anchors = a_ref[...]
    inv_2temp = 1.0 / (2.0 * jnp.maximum(rt_ref[0], 0.1))

    zt = jax.lax.dot_general(
        w, x, (((1,), (1,)), ((), ())),
        preferred_element_type=jnp.float32)

    mean = jnp.mean(zt, axis=0, keepdims=True)
    var = jnp.mean((zt - mean) * (zt - mean), axis=0, keepdims=True)
    scale = jnp.sqrt(var) + 1e-6
    qt = jnp.tanh(zt / (QUANT_TEMP * scale))
    q_ref[...] = qt

    dott = jax.lax.dot_general(
        anchors, qt, (((1,), (0,)), ((), ())),
        preferred_element_type=jnp.float32)

    logits = (dott - 6.0) * inv_2temp
    eidx = jax.lax.broadcasted_iota(jnp.int32, logits.shape, 0)

    m1 = jnp.max(logits, axis=0, keepdims=True)
    i1 = jnp.min(jnp.where(logits == m1, eidx, N_EXPERTS),
                 axis=0, keepdims=True)
    is1 = eidx == i1
    rest = jnp.where(is1, -jnp.inf, logits)
    m2 = jnp.max(rest, axis=0, keepdims=True)
    i2 = jnp.min(jnp.where(rest == m2, eidx, N_EXPERTS),
                 axis=0, keepdims=True)

    e2 = jnp.exp(m2 - m1)
    denom = 1.0 + e2
    w1 = 1.0 / denom
    w2 = e2 / denom
    ew_ref[...] = jnp.where(is1, w1, 0.0) + jnp.where(eidx == i2, w2, 0.0)


@jax.jit
def kernel(x, W_proj, routing_temp, expert_anchors):
    B, T, D = x.shape
    n_tok = B * T
    x2 = x.reshape(n_tok, D)
    rt = routing_temp.astype(jnp.float32).reshape(1)

    grid = (n_tok // BLK,)
    ewt, qt = pl.pallas_call(
        _router_kernel,
        grid=grid,
        in_specs=[
            pl.BlockSpec((BLK, D), lambda i: (i, 0)),
            pl.BlockSpec((K_PROJ, D), lambda i: (0, 0)),
            pl.BlockSpec((N_EXPERTS, K_PROJ), lambda i: (0, 0)),
            pl.BlockSpec(memory_space=pltpu.SMEM),
        ],
        out_specs=[
            pl.BlockSpec((N_EXPERTS, BLK), lambda i: (0, i)),
            pl.BlockSpec((K_PROJ, BLK), lambda i: (0, i)),
        ],
        out_shape=[
            jax.ShapeDtypeStruct((N_EXPERTS, n_tok), x.dtype),
            jax.ShapeDtypeStruct((K_PROJ, n_tok), jnp.float32),
        ],
        compiler_params=pltpu.CompilerParams(
            dimension_semantics=("arbitrary",)),
    )(x2, W_proj, expert_anchors, rt)

    return ewt.T.reshape(B, T, N_EXPERTS), qt.T.reshape(B, T, K_PROJ)
